# baseline (device time: 32810 ns/iter reference)
import os

_PASS = "tpu-custom-call-memory-space-policy"
_FLAGS = os.environ.get("XLA_FLAGS", "")
if _PASS not in _FLAGS:
    os.environ["XLA_FLAGS"] = (
        _FLAGS + " --xla_disable_hlo_passes=" + _PASS
    ).strip()

import jax
import jax.numpy as jnp
from jax import lax
from jax.experimental import pallas as pl
from jax.experimental.pallas import tpu as pltpu

N_LAYERS = 3
F32 = jnp.float32
BF16 = jnp.bfloat16


def kernel(x, Win0, Wout0, Win1, Wout1, Win2, Wout2):
    b, d_y = x.shape
    k_y, h_x = Win0.shape
    hc = h_x // 2
    oc = d_y // 2

    def body(x_hbm, win0_hbm, wout0_hbm, win1_hbm, wout1_hbm, win2_hbm,
             wout2_hbm, out_ref,
             x_v, win_v, wout_v,
             h_send, h_recv, o_send, o_recv,
             load_sems, send_sems, recv_sems):
        my_x = lax.axis_index("x")
        my_y = lax.axis_index("y")
        y_peer = (my_x, 1 - my_y)
        x_peer = (1 - my_x, my_y)

        ld_x = pltpu.make_async_copy(x_hbm, x_v, load_sems.at[0])
        ld_x.start()
        win_hbms = [win0_hbm, win1_hbm, win2_hbm]
        wout_hbms = [wout0_hbm, wout1_hbm, wout2_hbm]
        ld_win, ld_wout = [], []
        for i in range(N_LAYERS):
            lw = pltpu.make_async_copy(
                win_hbms[i], win_v.at[i], load_sems.at[1 + 2 * i])
            lw.start()
            ld_win.append(lw)
            lo = pltpu.make_async_copy(
                wout_hbms[i], wout_v.at[i], load_sems.at[2 + 2 * i])
            lo.start()
            ld_wout.append(lo)

        barrier = pltpu.get_barrier_semaphore()
        for peer in (y_peer, x_peer):
            pl.semaphore_signal(
                barrier, inc=1,
                device_id=peer, device_id_type=pl.DeviceIdType.MESH,
            )
        pl.semaphore_wait(barrier, 2)

        def exchange(src, dst, sem_idx, peer):
            r = pltpu.make_async_remote_copy(
                src_ref=src, dst_ref=dst,
                send_sem=send_sems.at[sem_idx],
                recv_sem=recv_sems.at[sem_idx],
                device_id=peer, device_id_type=pl.DeviceIdType.MESH,
            )
            r.start()
            return r

        ld_x.wait()
        ld_win[0].wait()
        xf = x_v[:, :]
        h_rdmas = []
        for c in range(2):
            h_send[c, :, :] = jnp.dot(
                xf, win_v[0, :, c * hc:(c + 1) * hc],
                preferred_element_type=F32,
            ).astype(BF16)
            h_rdmas.append(exchange(h_send.at[c], h_recv.at[c], c, y_peer))

        for l in range(N_LAYERS):
            hb = 2 * l
            sb = 4 * l

            ld_wout[l].wait()
            h_rdmas[0].wait()
            r0 = jnp.maximum(h_send[hb] + h_recv[hb], 0).astype(F32)
            t = [
                jnp.dot(r0, wout_v[l, :hc, c * oc:(c + 1) * oc],
                        preferred_element_type=F32)
                for c in range(2)
            ]
            h_rdmas[1].wait()
            r1 = jnp.maximum(h_send[hb + 1] + h_recv[hb + 1], 0).astype(F32)
            o_rdmas = []
            for c in range(2):
                po = t[c] + jnp.dot(
                    r1, wout_v[l, hc:, c * oc:(c + 1) * oc],
                    preferred_element_type=F32,
                )
                o_send[hb + c, :, :] = po.astype(BF16)
                o_rdmas.append(
                    exchange(o_send.at[hb + c], o_recv.at[hb + c],
                             sb + 2 + c, x_peer)
                )

            if l < N_LAYERS - 1:
                ld_win[l + 1].wait()
                o_rdmas[0].wait()
                xn0 = (o_send[hb] + o_recv[hb]).astype(F32)
                a = [
                    jnp.dot(xn0, win_v[l + 1, :oc, c * hc:(c + 1) * hc],
                            preferred_element_type=F32)
                    for c in range(2)
                ]
                o_rdmas[1].wait()
                xn1 = (o_send[hb + 1] + o_recv[hb + 1]).astype(F32)
                h_rdmas = []
                for c in range(2):
                    ph = a[c] + jnp.dot(
                        xn1, win_v[l + 1, oc:, c * hc:(c + 1) * hc],
                        preferred_element_type=F32,
                    )
                    h_send[2 * (l + 1) + c, :, :] = ph.astype(BF16)
                    h_rdmas.append(
                        exchange(h_send.at[2 * (l + 1) + c],
                                 h_recv.at[2 * (l + 1) + c],
                                 4 * (l + 1) + c, y_peer)
                    )
            else:
                o_rdmas[0].wait()
                o_rdmas[1].wait()
                out_ref[:, :oc] = (
                    o_send[hb].astype(F32) + o_recv[hb].astype(F32)
                )
                out_ref[:, oc:] = (
                    o_send[hb + 1].astype(F32) + o_recv[hb + 1].astype(F32)
                )

    return pl.pallas_call(
        body,
        out_shape=jax.ShapeDtypeStruct((b, d_y), jnp.float32),
        in_specs=[pl.BlockSpec(memory_space=pl.ANY)] * 7,
        out_specs=pl.BlockSpec(memory_space=pltpu.VMEM),
        scratch_shapes=[
            pltpu.VMEM((b, d_y), F32),
            pltpu.VMEM((N_LAYERS, k_y, h_x), F32),
            pltpu.VMEM((N_LAYERS, h_x, d_y), F32),
            pltpu.VMEM((2 * N_LAYERS, b, hc), BF16),
            pltpu.VMEM((2 * N_LAYERS, b, hc), BF16),
            pltpu.VMEM((2 * N_LAYERS, b, oc), BF16),
            pltpu.VMEM((2 * N_LAYERS, b, oc), BF16),
            pltpu.SemaphoreType.DMA((2 + 2 * N_LAYERS,)),
            pltpu.SemaphoreType.DMA((4 * N_LAYERS,)),
            pltpu.SemaphoreType.DMA((4 * N_LAYERS,)),
        ],
        compiler_params=pltpu.CompilerParams(collective_id=0),
    )(x, Win0, Wout0, Win1, Wout1, Win2, Wout2)


# device time: 31710 ns/iter; 1.0347x vs baseline; 1.0347x over previous
import jax
import jax.numpy as jnp
from jax import lax
from jax.experimental import pallas as pl
from jax.experimental.pallas import tpu as pltpu

N_LAYERS = 3
F32 = jnp.float32
BF16 = jnp.bfloat16


def kernel(x, Win0, Wout0, Win1, Wout1, Win2, Wout2):
    b, d_y = x.shape
    k_y, h_x = Win0.shape
    hc = h_x // 2
    oc = d_y // 2

    def body(x_ref, win0_ref, wout0_ref, win1_ref, wout1_ref, win2_ref,
             wout2_ref, out_ref,
             h_send, h_recv, o_send, o_recv, send_sems, recv_sems):
        my_x = lax.axis_index("x")
        my_y = lax.axis_index("y")
        y_peer = (my_x, 1 - my_y)
        x_peer = (1 - my_x, my_y)

        barrier = pltpu.get_barrier_semaphore()
        for peer in (y_peer, x_peer):
            pl.semaphore_signal(
                barrier, inc=1,
                device_id=peer, device_id_type=pl.DeviceIdType.MESH,
            )
        pl.semaphore_wait(barrier, 2)

        wins = [win0_ref, win1_ref, win2_ref]
        wouts = [wout0_ref, wout1_ref, wout2_ref]

        def exchange(src, dst, sem_idx, peer):
            r = pltpu.make_async_remote_copy(
                src_ref=src, dst_ref=dst,
                send_sem=send_sems.at[sem_idx],
                recv_sem=recv_sems.at[sem_idx],
                device_id=peer, device_id_type=pl.DeviceIdType.MESH,
            )
            r.start()
            return r

        xf = x_ref[:, :]
        h_rdmas = []
        for c in range(2):
            h_send[c, :, :] = jnp.dot(
                xf, wins[0][:, c * hc:(c + 1) * hc],
                preferred_element_type=F32,
            ).astype(BF16)
            h_rdmas.append(exchange(h_send.at[c], h_recv.at[c], c, y_peer))

        for l in range(N_LAYERS):
            hb = 2 * l
            sb = 4 * l
            wout = wouts[l]

            h_rdmas[0].wait()
            r0 = jnp.maximum(h_send[hb] + h_recv[hb], 0).astype(F32)
            t = [
                jnp.dot(r0, wout[:hc, c * oc:(c + 1) * oc],
                        preferred_element_type=F32)
                for c in range(2)
            ]
            h_rdmas[1].wait()
            r1 = jnp.maximum(h_send[hb + 1] + h_recv[hb + 1], 0).astype(F32)
            o_rdmas = []
            for c in range(2):
                po = t[c] + jnp.dot(
                    r1, wout[hc:, c * oc:(c + 1) * oc],
                    preferred_element_type=F32,
                )
                o_send[hb + c, :, :] = po.astype(BF16)
                o_rdmas.append(
                    exchange(o_send.at[hb + c], o_recv.at[hb + c],
                             sb + 2 + c, x_peer)
                )

            if l < N_LAYERS - 1:
                win_n = wins[l + 1]
                o_rdmas[0].wait()
                xn0 = (o_send[hb] + o_recv[hb]).astype(F32)
                a = [
                    jnp.dot(xn0, win_n[:oc, c * hc:(c + 1) * hc],
                            preferred_element_type=F32)
                    for c in range(2)
                ]
                o_rdmas[1].wait()
                xn1 = (o_send[hb + 1] + o_recv[hb + 1]).astype(F32)
                h_rdmas = []
                for c in range(2):
                    ph = a[c] + jnp.dot(
                        xn1, win_n[oc:, c * hc:(c + 1) * hc],
                        preferred_element_type=F32,
                    )
                    h_send[2 * (l + 1) + c, :, :] = ph.astype(BF16)
                    h_rdmas.append(
                        exchange(h_send.at[2 * (l + 1) + c],
                                 h_recv.at[2 * (l + 1) + c],
                                 4 * (l + 1) + c, y_peer)
                    )
            else:
                o_rdmas[0].wait()
                o_rdmas[1].wait()
                out_ref[:, :oc] = (
                    o_send[hb].astype(F32) + o_recv[hb].astype(F32)
                )
                out_ref[:, oc:] = (
                    o_send[hb + 1].astype(F32) + o_recv[hb + 1].astype(F32)
                )

    return pl.pallas_call(
        body,
        out_shape=jax.ShapeDtypeStruct((b, d_y), jnp.float32),
        in_specs=[pl.BlockSpec(memory_space=pltpu.VMEM)] * 7,
        out_specs=pl.BlockSpec(memory_space=pltpu.VMEM),
        scratch_shapes=[
            pltpu.VMEM((2 * N_LAYERS, b, hc), BF16),
            pltpu.VMEM((2 * N_LAYERS, b, hc), BF16),
            pltpu.VMEM((2 * N_LAYERS, b, oc), BF16),
            pltpu.VMEM((2 * N_LAYERS, b, oc), BF16),
            pltpu.SemaphoreType.DMA((4 * N_LAYERS,)),
            pltpu.SemaphoreType.DMA((4 * N_LAYERS,)),
        ],
        compiler_params=pltpu.CompilerParams(collective_id=0),
    )(x, Win0, Wout0, Win1, Wout1, Win2, Wout2)
